# baseline (speedup 1.0000x reference)
import functools

import jax
import jax.numpy as jnp
from jax import lax
from jax.experimental import pallas as pl
from jax.experimental.pallas import tpu as pltpu
from jax.experimental.pallas import tpu_sc as plsc

D = 64
SCALE = 8.0
NC = 2
NS = 16
NW = NC * NS
NBUF = 4


@functools.lru_cache(maxsize=None)
def _make(S: int, L: int, V: int):
    assert S % NW == 0
    G = S // NW
    LP = -(-L // 8) * 8
    mesh = plsc.VectorSubcoreMesh(core_axis_name="c", subcore_axis_name="s")

    @functools.partial(
        pl.kernel,
        mesh=mesh,
        out_type=jax.ShapeDtypeStruct((S, L, D), jnp.float32),
        compiler_params=pltpu.CompilerParams(use_tc_tiling_on_sc=False),
        scratch_types=[
            pltpu.VMEM((G, LP), jnp.int32),
            *[pltpu.VMEM((LP, D), jnp.float32) for _ in range(NBUF)],
            *[pltpu.SemaphoreType.DMA for _ in range(2 * NBUF)],
        ],
    )
    def emb(x_hbm, lut_hbm, out_hbm, idx_v, r0, r1, r2, r3,
            g0, g1, g2, g3, s0, s1, s2, s3):
        bufs = (r0, r1, r2, r3)
        gsem = (g0, g1, g2, g3)
        ssem = (s0, s1, s2, s3)
        wid = lax.axis_index("s") * NC + lax.axis_index("c")
        base = wid * G

        pltpu.sync_copy(x_hbm.at[wid], idx_v)

        def start_gather(g, b):
            pltpu.async_copy(lut_hbm.at[idx_v.at[g]], bufs[b], gsem[b])

        def wait_gather(g, b):
            pltpu.make_async_copy(lut_hbm.at[idx_v.at[g]], bufs[b],
                                  gsem[b]).wait()

        def start_scatter(g, b):
            pltpu.async_copy(bufs[b].at[pl.ds(0, L)], out_hbm.at[base + g],
                             ssem[b])

        def wait_scatter(g, b):
            pltpu.make_async_copy(bufs[b].at[pl.ds(0, L)],
                                  out_hbm.at[base + g], ssem[b]).wait()

        start_gather(0, 0)
        start_gather(1, 1)

        def body(i, carry):
            for b in range(NBUF):
                g = i * NBUF + b
                bn = (b + 2) % NBUF
                pl.when(g >= 2)(lambda: wait_scatter(g - 2, bn))
                pl.when(g + 2 < G)(lambda: start_gather(g + 2, bn))
                wait_gather(g, b)
                start_scatter(g, b)
            return carry

        lax.fori_loop(0, G // NBUF, body, 0)

        wait_scatter(G - 2, (G - 2) % NBUF)
        wait_scatter(G - 1, (G - 1) % NBUF)

    return emb


def kernel(x, lut):
    S, L = x.shape
    V = lut.shape[0]
    lut8 = lut * SCALE
    LP = -(-L // 8) * 8
    x3 = x.astype(jnp.int32).reshape(NW, S // NW, L)
    x3 = jnp.pad(x3, ((0, 0), (0, 0), (0, LP - L)))
    return _make(S, L, V)(x3, lut8)
